# Initial kernel scaffold; baseline (speedup 1.0000x reference)
import jax
import jax.numpy as jnp
from jax.experimental import pallas as pl


def kernel(x, e_f, params, edge_index):
    raise NotImplementedError("write your pallas kernel here")


# speedup vs baseline: 1.1721x; 1.1721x over previous
import functools

import jax
import jax.numpy as jnp
from jax.experimental import pallas as pl

H = 128
HEADS = 4
NEG_SLOPE = 0.2


def _mm_kernel(x_ref, w_ref, o_ref):
    o_ref[...] = jnp.dot(x_ref[...], w_ref[...],
                         preferred_element_type=jnp.float32)


def _mm(x, w, bm):
    M, K = x.shape
    _, N = w.shape
    return pl.pallas_call(
        _mm_kernel,
        grid=(M // bm,),
        in_specs=[pl.BlockSpec((bm, K), lambda i: (i, 0)),
                  pl.BlockSpec((K, N), lambda i: (0, 0))],
        out_specs=pl.BlockSpec((bm, N), lambda i: (i, 0)),
        out_shape=jax.ShapeDtypeStruct((M, N), jnp.float32),
    )(x, w)


def _msg_kernel(u_ref, we_ref, ex_ref, o_ref):
    ex = ex_ref[...]
    exw = jnp.repeat(ex, H, axis=1)
    o_ref[...] = u_ref[...] * we_ref[...] * exw


def _msg(u, we, ex, bm):
    E = u.shape[0]
    return pl.pallas_call(
        _msg_kernel,
        grid=(E // bm,),
        in_specs=[pl.BlockSpec((bm, HEADS * H), lambda i: (i, 0)),
                  pl.BlockSpec((bm, HEADS * H), lambda i: (i, 0)),
                  pl.BlockSpec((bm, HEADS), lambda i: (i, 0))],
        out_specs=pl.BlockSpec((bm, HEADS * H), lambda i: (i, 0)),
        out_shape=jax.ShapeDtypeStruct((E, HEADS * H), jnp.float32),
    )(u, we, ex)


def _combine_kernel(agg_ref, deg_ref, h_ref, ws_ref, b_ref, o_ref):
    out = jnp.dot(agg_ref[...], ws_ref[...],
                  preferred_element_type=jnp.float32) + b_ref[...]
    mask = deg_ref[...] > 0.0
    o_ref[...] = h_ref[...] + jnp.where(mask, out, 0.0)


def _combine(agg, deg, h, w_scale, bias, bm):
    N = agg.shape[0]
    return pl.pallas_call(
        _combine_kernel,
        grid=(N // bm,),
        in_specs=[pl.BlockSpec((bm, HEADS * H), lambda i: (i, 0)),
                  pl.BlockSpec((bm, 1), lambda i: (i, 0)),
                  pl.BlockSpec((bm, H), lambda i: (i, 0)),
                  pl.BlockSpec((HEADS * H, H), lambda i: (0, 0)),
                  pl.BlockSpec((1, H), lambda i: (0, 0))],
        out_specs=pl.BlockSpec((bm, H), lambda i: (i, 0)),
        out_shape=jax.ShapeDtypeStruct((N, H), jnp.float32),
    )(agg, deg, h, w_scale, bias)


def _set2set_kernel(feat_ref, wih_ref, whh_ref, bsum_ref, wlin_ref,
                    blin_ref, o_ref):
    feat = feat_ref[...]
    q_star = jnp.zeros((1, 2 * H), jnp.float32)
    hh = jnp.zeros((1, H), jnp.float32)
    cc = jnp.zeros((1, H), jnp.float32)
    for _ in range(2):
        gates = (jnp.dot(q_star, wih_ref[...],
                         preferred_element_type=jnp.float32)
                 + jnp.dot(hh, whh_ref[...],
                           preferred_element_type=jnp.float32)
                 + bsum_ref[...])
        i = jax.nn.sigmoid(gates[:, 0 * H:1 * H])
        f = jax.nn.sigmoid(gates[:, 1 * H:2 * H])
        g = jnp.tanh(gates[:, 2 * H:3 * H])
        o = jax.nn.sigmoid(gates[:, 3 * H:4 * H])
        cc = f * cc + i * g
        hh = o * jnp.tanh(cc)
        escore = jnp.sum(feat * hh, axis=-1, keepdims=True)
        m = jnp.max(escore, axis=0, keepdims=True)
        ea = jnp.exp(escore - m)
        alpha = ea / jnp.sum(ea, axis=0, keepdims=True)
        readout = jnp.sum(alpha * feat, axis=0, keepdims=True)
        q_star = jnp.concatenate([hh, readout], axis=-1)
    o_ref[...] = jnp.dot(q_star, wlin_ref[...],
                         preferred_element_type=jnp.float32) + blin_ref[...]


def _set2set(feat, lstm, w_lin, b_lin):
    N = feat.shape[0]
    wih = lstm["W_ih"].T
    whh = lstm["W_hh"].T
    bsum = (lstm["b_ih"] + lstm["b_hh"]).reshape(1, 4 * H)
    return pl.pallas_call(
        _set2set_kernel,
        in_specs=[pl.BlockSpec((N, H), lambda: (0, 0)),
                  pl.BlockSpec((2 * H, 4 * H), lambda: (0, 0)),
                  pl.BlockSpec((H, 4 * H), lambda: (0, 0)),
                  pl.BlockSpec((1, 4 * H), lambda: (0, 0)),
                  pl.BlockSpec((2 * H, H), lambda: (0, 0)),
                  pl.BlockSpec((1, H), lambda: (0, 0))],
        out_specs=pl.BlockSpec((1, H), lambda: (0, 0)),
        out_shape=jax.ShapeDtypeStruct((1, H), jnp.float32),
    )(feat, wih, whh, bsum, w_lin.reshape(2 * H, H), b_lin.reshape(1, H))


def _layer(h, e, src, dst, p):
    N = h.shape[0]
    w1 = p["w_att"][0, :, :H]
    w2 = p["w_att"][0, :, H:2 * H]
    w3 = p["w_att"][0, :, 2 * H:]
    wn = p["W_node"].reshape(H, HEADS, H)
    wedge = p["W_edge"].reshape(H, HEADS, H)
    wss = jnp.einsum('khd,hd->kh', wn, w1)
    wsd = jnp.einsum('khd,hd->kh', wn, w3)
    wse = jnp.einsum('khd,hd->kh', wedge, w2)

    wv = _mm(h, p["W_node"], 400)
    we = _mm(e, p["W_edge"], 2000)
    s_nodes = _mm(h, jnp.concatenate([wss, wsd], axis=1), 400)
    s_e = _mm(e, wse, 2000)

    s_src = s_nodes[:, :HEADS]
    s_dst = s_nodes[:, HEADS:]
    attn = s_src[src] + s_e + s_dst[dst]
    attn = jnp.where(attn >= 0, attn, NEG_SLOPE * attn)
    amax = jax.ops.segment_max(attn, dst, num_segments=N)
    amax = jnp.where(jnp.isfinite(amax), amax, 0.0)
    ex = jnp.exp(attn - amax[dst])
    denom = jax.ops.segment_sum(ex, dst, num_segments=N)
    score = ex / jnp.maximum(denom[dst], 1e-16)

    u = wv[src]
    msg = _msg(u, we, score, 2000)
    agg = jax.ops.segment_sum(msg, dst, num_segments=N)
    deg = jax.ops.segment_sum(jnp.ones((dst.shape[0],), jnp.float32),
                              dst, num_segments=N)
    return _combine(agg, deg.reshape(N, 1), h, p["W_scale"],
                    p["bias"].reshape(1, H), 400)


@jax.jit
def kernel(x, e_f, params, edge_index):
    src = edge_index[0]
    dst = edge_index[1]
    h = _mm(x, params["W_proj"], 400)
    e = _mm(e_f, params["W_proj_e"], 2000)
    for p in params["convs"]:
        h = _layer(h, e, src, dst, p)
    return _set2set(h, params["lstm"], params["W_lin"], params["b_lin"])


# speedup vs baseline: 1.4756x; 1.2589x over previous
import functools

import jax
import jax.numpy as jnp
from jax.experimental import pallas as pl

H = 128
HEADS = 4
NEG_SLOPE = 0.2


def _mm_kernel(x_ref, w_ref, o_ref):
    o_ref[...] = jnp.dot(x_ref[...], w_ref[...],
                         preferred_element_type=jnp.float32)


def _mm(x, w, bm):
    M, K = x.shape
    _, N = w.shape
    return pl.pallas_call(
        _mm_kernel,
        grid=(M // bm,),
        in_specs=[pl.BlockSpec((bm, K), lambda i: (i, 0)),
                  pl.BlockSpec((K, N), lambda i: (0, 0))],
        out_specs=pl.BlockSpec((bm, N), lambda i: (i, 0)),
        out_shape=jax.ShapeDtypeStruct((M, N), jnp.float32),
    )(x, w)


def _bound_kernel(sn_ref, se_ref, o_ref):
    i = pl.program_id(0)
    nblk = pl.num_programs(0)
    bmax = jnp.max(se_ref[...], axis=0, keepdims=True)

    @pl.when(i == 0)
    def _():
        o_ref[...] = bmax

    @pl.when(i > 0)
    def _():
        o_ref[...] = jnp.maximum(o_ref[...], bmax)

    @pl.when(i == nblk - 1)
    def _():
        sn = sn_ref[...]
        m = (o_ref[...]
             + jnp.max(sn[:, :HEADS], axis=0, keepdims=True)
             + jnp.max(sn[:, HEADS:], axis=0, keepdims=True))
        o_ref[...] = jnp.where(m >= 0, m, NEG_SLOPE * m)


def _bound(s_nodes, s_e, bm):
    N = s_nodes.shape[0]
    E = s_e.shape[0]
    return pl.pallas_call(
        _bound_kernel,
        grid=(E // bm,),
        in_specs=[pl.BlockSpec((N, 2 * HEADS), lambda i: (0, 0)),
                  pl.BlockSpec((bm, HEADS), lambda i: (i, 0))],
        out_specs=pl.BlockSpec((1, HEADS), lambda i: (0, 0)),
        out_shape=jax.ShapeDtypeStruct((1, HEADS), jnp.float32),
    )(s_nodes, s_e)


def _msg_kernel(u_ref, we_ref, a_ref, m_ref, o_ref, ex_ref):
    a = a_ref[...]
    a = jnp.where(a >= 0, a, NEG_SLOPE * a)
    ex = jnp.exp(a - m_ref[...])
    ex_ref[...] = ex
    exw = jnp.repeat(ex, H, axis=1)
    o_ref[...] = u_ref[...] * we_ref[...] * exw


def _msg(u, we, attn_raw, m, bm):
    E = u.shape[0]
    return pl.pallas_call(
        _msg_kernel,
        grid=(E // bm,),
        in_specs=[pl.BlockSpec((bm, HEADS * H), lambda i: (i, 0)),
                  pl.BlockSpec((bm, HEADS * H), lambda i: (i, 0)),
                  pl.BlockSpec((bm, HEADS), lambda i: (i, 0)),
                  pl.BlockSpec((1, HEADS), lambda i: (0, 0))],
        out_specs=[pl.BlockSpec((bm, HEADS * H), lambda i: (i, 0)),
                   pl.BlockSpec((bm, HEADS), lambda i: (i, 0))],
        out_shape=[jax.ShapeDtypeStruct((E, HEADS * H), jnp.float32),
                   jax.ShapeDtypeStruct((E, HEADS), jnp.float32)],
    )(u, we, attn_raw, m)


def _combine_kernel(agg_ref, den_ref, h_ref, ws_ref, b_ref, o_ref):
    den = den_ref[...]
    inv = 1.0 / jnp.maximum(den, 1e-16)
    agg = agg_ref[...] * jnp.repeat(inv, H, axis=1)
    out = jnp.dot(agg, ws_ref[...],
                  preferred_element_type=jnp.float32) + b_ref[...]
    mask = den[:, :1] > 0.0
    o_ref[...] = h_ref[...] + jnp.where(mask, out, 0.0)


def _combine(agg, den, h, w_scale, bias, bm):
    N = agg.shape[0]
    return pl.pallas_call(
        _combine_kernel,
        grid=(N // bm,),
        in_specs=[pl.BlockSpec((bm, HEADS * H), lambda i: (i, 0)),
                  pl.BlockSpec((bm, HEADS), lambda i: (i, 0)),
                  pl.BlockSpec((bm, H), lambda i: (i, 0)),
                  pl.BlockSpec((HEADS * H, H), lambda i: (0, 0)),
                  pl.BlockSpec((1, H), lambda i: (0, 0))],
        out_specs=pl.BlockSpec((bm, H), lambda i: (i, 0)),
        out_shape=jax.ShapeDtypeStruct((N, H), jnp.float32),
    )(agg, den, h, w_scale, bias)


def _set2set_kernel(feat_ref, wih_ref, whh_ref, bsum_ref, wlin_ref,
                    blin_ref, o_ref):
    feat = feat_ref[...]
    q_star = jnp.zeros((1, 2 * H), jnp.float32)
    hh = jnp.zeros((1, H), jnp.float32)
    cc = jnp.zeros((1, H), jnp.float32)
    for _ in range(2):
        gates = (jnp.dot(q_star, wih_ref[...],
                         preferred_element_type=jnp.float32)
                 + jnp.dot(hh, whh_ref[...],
                           preferred_element_type=jnp.float32)
                 + bsum_ref[...])
        i = jax.nn.sigmoid(gates[:, 0 * H:1 * H])
        f = jax.nn.sigmoid(gates[:, 1 * H:2 * H])
        g = jnp.tanh(gates[:, 2 * H:3 * H])
        o = jax.nn.sigmoid(gates[:, 3 * H:4 * H])
        cc = f * cc + i * g
        hh = o * jnp.tanh(cc)
        escore = jnp.sum(feat * hh, axis=-1, keepdims=True)
        m = jnp.max(escore, axis=0, keepdims=True)
        ea = jnp.exp(escore - m)
        alpha = ea / jnp.sum(ea, axis=0, keepdims=True)
        readout = jnp.sum(alpha * feat, axis=0, keepdims=True)
        q_star = jnp.concatenate([hh, readout], axis=-1)
    o_ref[...] = jnp.dot(q_star, wlin_ref[...],
                         preferred_element_type=jnp.float32) + blin_ref[...]


def _set2set(feat, lstm, w_lin, b_lin):
    N = feat.shape[0]
    wih = lstm["W_ih"].T
    whh = lstm["W_hh"].T
    bsum = (lstm["b_ih"] + lstm["b_hh"]).reshape(1, 4 * H)
    return pl.pallas_call(
        _set2set_kernel,
        in_specs=[pl.BlockSpec((N, H), lambda: (0, 0)),
                  pl.BlockSpec((2 * H, 4 * H), lambda: (0, 0)),
                  pl.BlockSpec((H, 4 * H), lambda: (0, 0)),
                  pl.BlockSpec((1, 4 * H), lambda: (0, 0)),
                  pl.BlockSpec((2 * H, H), lambda: (0, 0)),
                  pl.BlockSpec((1, H), lambda: (0, 0))],
        out_specs=pl.BlockSpec((1, H), lambda: (0, 0)),
        out_shape=jax.ShapeDtypeStruct((1, H), jnp.float32),
    )(feat, wih, whh, bsum, w_lin.reshape(2 * H, H), b_lin.reshape(1, H))


def _layer(h, e, src, dst, p):
    N = h.shape[0]
    w1 = p["w_att"][0, :, :H]
    w2 = p["w_att"][0, :, H:2 * H]
    w3 = p["w_att"][0, :, 2 * H:]
    wn = p["W_node"].reshape(H, HEADS, H)
    wedge = p["W_edge"].reshape(H, HEADS, H)
    wss = jnp.einsum('khd,hd->kh', wn, w1)
    wsd = jnp.einsum('khd,hd->kh', wn, w3)
    wse = jnp.einsum('khd,hd->kh', wedge, w2)

    wv = _mm(h, p["W_node"], 400)
    we = _mm(e, p["W_edge"], 2000)
    s_nodes = _mm(h, jnp.concatenate([wss, wsd], axis=1), 400)
    s_e = _mm(e, wse, 2000)
    m = _bound(s_nodes, s_e, 2000)

    s_src = s_nodes[:, :HEADS]
    s_dst = s_nodes[:, HEADS:]
    attn_raw = s_src[src] + s_e + s_dst[dst]

    u = wv[src]
    msg, ex = _msg(u, we, attn_raw, m, 2000)
    agg = jax.ops.segment_sum(msg, dst, num_segments=N)
    den = jax.ops.segment_sum(ex, dst, num_segments=N)
    return _combine(agg, den, h, p["W_scale"],
                    p["bias"].reshape(1, H), 400)


@jax.jit
def kernel(x, e_f, params, edge_index):
    src = edge_index[0]
    dst = edge_index[1]
    h = _mm(x, params["W_proj"], 400)
    e = _mm(e_f, params["W_proj_e"], 2000)
    for p in params["convs"]:
        h = _layer(h, e, src, dst, p)
    return _set2set(h, params["lstm"], params["W_lin"], params["b_lin"])
